# baseline (speedup 1.0000x reference)
import functools

import jax
import jax.numpy as jnp
from jax import lax
from jax.experimental import pallas as pl
from jax.experimental.pallas import tpu as pltpu
from jax.experimental.pallas import tpu_sc as plsc

_EPS = 1e-5


def _sc_gather(t2, t1r, idx):
  n = idx.shape[1]
  e = t2.shape[1]
  w = 128
  mesh = plsc.VectorSubcoreMesh(core_axis_name="core",
                                subcore_axis_name="subcore")

  @functools.partial(
      pl.kernel,
      out_type=(jax.ShapeDtypeStruct((n, e), jnp.float32),
                jax.ShapeDtypeStruct((n,), jnp.float32)),
      mesh=mesh,
      compiler_params=pltpu.CompilerParams(use_tc_tiling_on_sc=False,
                                           needs_layout_passes=False))
  def k(t2_hbm, t1r_hbm, i_hbm, o2_hbm, o1_hbm):
    def body(i_vmem, o2_vmem, o1_vmem):
      pltpu.sync_copy(t2_hbm.at[i_vmem.at[0]], o2_vmem)

      def inner(row_idx, rows):
        iv = i_vmem.at[0]

        @pl.loop(0, w, step=16)
        def _(j):
          row_idx[pl.ds(j, 16)] = jax.lax.shift_right_logical(
              iv[pl.ds(j, 16)], 4)

        pltpu.sync_copy(t1r_hbm.at[row_idx], rows)

        @pl.loop(0, w, step=16)
        def _(j):
          lanes = jax.lax.bitwise_and(iv[pl.ds(j, 16)], 15)
          ks = jax.lax.iota(jnp.int32, 16) + j
          o1_vmem[pl.ds(j, 16)] = plsc.load_gather(rows, [ks, lanes])

      pl.run_scoped(inner,
                    row_idx=pltpu.VMEM((w,), jnp.int32),
                    rows=pltpu.VMEM((w, 16), jnp.float32))

    pltpu.emit_pipeline(
        body,
        grid=(n // w,),
        in_specs=[pl.BlockSpec((1, w), lambda i: (0, i))],
        out_specs=[pl.BlockSpec((w, e), lambda i: (i, 0)),
                   pl.BlockSpec((w,), lambda i: (i,))],
        core_axis_name=("core", "subcore"),
        dimension_semantics=(pltpu.PARALLEL,),
    )(i_hbm, o2_hbm, o1_hbm)

  return k(t2, t1r, idx)


def _stage_a_body(o2f_ref, o1v_ref, dense_ref, wdl_ref, bdl_ref, w1d_ref,
                  b1d_ref, wl1_ref, bl1_ref, h1_ref, st_ref):
  i = pl.program_id(0)
  fe = o2f_ref.shape[1]
  e = fe // o1v_ref.shape[1]
  o2 = o2f_ref[...]
  dense = dense_ref[...]
  sel = (lax.broadcasted_iota(jnp.int32, (fe, e), 0) % e ==
         lax.broadcasted_iota(jnp.int32, (fe, e), 1)).astype(jnp.float32)
  sum_emb = jax.lax.dot(o2, sel)
  order2 = 0.5 * (jnp.sum(sum_emb * sum_emb, axis=1, keepdims=True)
                  - jnp.sum(o2 * o2, axis=1, keepdims=True))
  order1 = (jnp.sum(o1v_ref[...], axis=1, keepdims=True)
            + jax.lax.dot(dense, w1d_ref[...]) + b1d_ref[...])
  dd = jnp.maximum(jax.lax.dot(dense, wdl_ref[...]) + bdl_ref[...], 0.0)
  dnn_in = o2 + dd + order1 + order2
  h1 = jax.lax.dot(dnn_in, wl1_ref[...]) + bl1_ref[...]
  h1_ref[...] = h1
  st = jnp.concatenate([jnp.sum(h1, axis=0, keepdims=True),
                        jnp.sum(h1 * h1, axis=0, keepdims=True)], axis=0)

  @pl.when(i == 0)
  def _():
    st_ref[...] = st

  @pl.when(i > 0)
  def _():
    st_ref[...] = st_ref[...] + st


def _stage_mid_body(x_ref, st_in_ref, g_ref, be_ref, w_ref, b_ref,
                    y_ref, st_ref, *, batch):
  i = pl.program_id(0)
  mean = st_in_ref[0:1, :] / batch
  var = st_in_ref[1:2, :] / batch - mean * mean
  a = g_ref[...] * lax.rsqrt(var + _EPS)
  c = be_ref[...] - mean * a
  x = jnp.maximum(x_ref[...] * a + c, 0.0)
  y = jax.lax.dot(x, w_ref[...]) + b_ref[...]
  y_ref[...] = y
  st = jnp.concatenate([jnp.sum(y, axis=0, keepdims=True),
                        jnp.sum(y * y, axis=0, keepdims=True)], axis=0)

  @pl.when(i == 0)
  def _():
    st_ref[...] = st

  @pl.when(i > 0)
  def _():
    st_ref[...] = st_ref[...] + st


def _stage_c_body(x_ref, st_in_ref, g_ref, be_ref, w_ref, b_ref, out_ref, *,
                  batch):
  mean = st_in_ref[0:1, :] / batch
  var = st_in_ref[1:2, :] / batch - mean * mean
  a = g_ref[...] * lax.rsqrt(var + _EPS)
  c = be_ref[...] - mean * a
  x = jnp.maximum(x_ref[...] * a + c, 0.0)
  out_ref[...] = jax.lax.dot(x, w_ref[...]) + b_ref[...]


def _full(shape):
  return pl.BlockSpec(shape, lambda i: (0, 0))


def kernel(sparse, dense, order1_emb, order2_emb, W1d, b1d, Wdl, bdl, W_l1,
           b_l1, g1, be1, W_l2, b_l2, g2, be2, W_out, b_out):
  b, f = sparse.shape
  fv, e = order2_emb.shape[1], order2_emb.shape[2]
  d = dense.shape[1]
  h1n, h2n = W_l1.shape[0], W_l2.shape[0]
  fe = f * e
  n = b * f
  bb = 1024
  grid = (b // bb,)

  t2 = order2_emb.reshape(f * fv, e)
  t1r = order1_emb.reshape(f * fv // 16, 16)
  idx = (sparse + (jnp.arange(f, dtype=jnp.int32) * fv)[None, :]).reshape(1, n)
  o2r, o1r = _sc_gather(t2, t1r, idx)
  o2f = o2r.reshape(b, fe)
  o1v = o1r.reshape(b, f)

  fl = jnp.float32
  h1, st1 = pl.pallas_call(
      _stage_a_body,
      grid=grid,
      in_specs=[
          pl.BlockSpec((bb, fe), lambda i: (i, 0)),
          pl.BlockSpec((bb, f), lambda i: (i, 0)),
          pl.BlockSpec((bb, d), lambda i: (i, 0)),
          _full((d, fe)), _full((1, fe)), _full((d, 1)), _full((1, 1)),
          _full((fe, h1n)), _full((1, h1n)),
      ],
      out_specs=[pl.BlockSpec((bb, h1n), lambda i: (i, 0)),
                 _full((2, h1n))],
      out_shape=(jax.ShapeDtypeStruct((b, h1n), fl),
                 jax.ShapeDtypeStruct((2, h1n), fl)),
  )(o2f, o1v, dense, Wdl.T, bdl.reshape(1, fe), W1d.T, b1d.reshape(1, 1),
    W_l1.T, b_l1.reshape(1, h1n))

  h2, st2 = pl.pallas_call(
      functools.partial(_stage_mid_body, batch=float(b)),
      grid=grid,
      in_specs=[
          pl.BlockSpec((bb, h1n), lambda i: (i, 0)),
          _full((2, h1n)), _full((1, h1n)), _full((1, h1n)),
          _full((h1n, h2n)), _full((1, h2n)),
      ],
      out_specs=[pl.BlockSpec((bb, h2n), lambda i: (i, 0)),
                 _full((2, h2n))],
      out_shape=(jax.ShapeDtypeStruct((b, h2n), fl),
                 jax.ShapeDtypeStruct((2, h2n), fl)),
  )(h1, st1, g1.reshape(1, h1n), be1.reshape(1, h1n), W_l2.T,
    b_l2.reshape(1, h2n))

  out = pl.pallas_call(
      functools.partial(_stage_c_body, batch=float(b)),
      grid=grid,
      in_specs=[
          pl.BlockSpec((bb, h2n), lambda i: (i, 0)),
          _full((2, h2n)), _full((1, h2n)), _full((1, h2n)),
          _full((h2n, 1)), _full((1, 1)),
      ],
      out_specs=pl.BlockSpec((bb, 1), lambda i: (i, 0)),
      out_shape=jax.ShapeDtypeStruct((b, 1), fl),
  )(h2, st2, g2.reshape(1, h2n), be2.reshape(1, h2n), W_out.T,
    b_out.reshape(1, 1))
  return out


# speedup vs baseline: 5.5162x; 5.5162x over previous
import functools

import jax
import jax.numpy as jnp
from jax import lax
from jax.experimental import pallas as pl
from jax.experimental.pallas import tpu as pltpu
from jax.experimental.pallas import tpu_sc as plsc

_EPS = 1e-5
_NW = 32
_CH = 8192


def _sc_gather(t2T, t1T, idxT):
  f, e, v = t2T.shape
  b = idxT.shape[1]
  mesh = plsc.VectorSubcoreMesh(core_axis_name="core",
                                subcore_axis_name="subcore")

  @functools.partial(
      pl.kernel,
      out_type=(jax.ShapeDtypeStruct((f, e, b), jnp.float32),
                jax.ShapeDtypeStruct((f, b), jnp.float32)),
      mesh=mesh,
      scratch_types=[pltpu.VMEM((v,), jnp.float32),
                     pltpu.VMEM((_CH,), jnp.int32),
                     pltpu.VMEM((_CH,), jnp.float32)],
      compiler_params=pltpu.CompilerParams(use_tc_tiling_on_sc=True,
                                           needs_layout_passes=False))
  def k(t2T_hbm, t1T_hbm, idxT_hbm, o2_hbm, o1_hbm, plane, idxv, outv):
    w = lax.axis_index("subcore") * 2 + lax.axis_index("core")

    @pl.loop(0, f * e // _NW)
    def _(pi):
      p = w + pi * _NW
      pf = p // e
      pe = p % e
      pltpu.sync_copy(t2T_hbm.at[pf, pe], plane)

      @pl.loop(0, b, step=_CH)
      def _(c):
        pltpu.sync_copy(idxT_hbm.at[pf, pl.ds(c, _CH)], idxv)

        @pl.loop(0, _CH, step=64)
        def _(j):
          for u in range(0, 64, 16):
            outv[pl.ds(j + u, 16)] = plsc.load_gather(
                plane, [idxv[pl.ds(j + u, 16)]])

        pltpu.sync_copy(outv, o2_hbm.at[pf, pe, pl.ds(c, _CH)])

    @pl.when(w < f)
    def _():
      pltpu.sync_copy(t1T_hbm.at[w, 0], plane)

      @pl.loop(0, b, step=_CH)
      def _(c):
        pltpu.sync_copy(idxT_hbm.at[w, pl.ds(c, _CH)], idxv)

        @pl.loop(0, _CH, step=64)
        def _(j):
          for u in range(0, 64, 16):
            outv[pl.ds(j + u, 16)] = plsc.load_gather(
                plane, [idxv[pl.ds(j + u, 16)]])

        pltpu.sync_copy(outv, o1_hbm.at[w, pl.ds(c, _CH)])

  return k(t2T, t1T, idxT)


def _stage_a_body(oT_ref, o1T_ref, dT_ref, wdl_ref, bdl_ref, w1d_ref, b1d_ref,
                  wl1_ref, bl1_ref, h1_ref, st_ref):
  i = pl.program_id(0)
  fe = oT_ref.shape[0]
  e = fe // o1T_ref.shape[0]
  oT = oT_ref[...]
  dT = dT_ref[...]
  sel = (lax.broadcasted_iota(jnp.int32, (e, fe), 0) ==
         lax.broadcasted_iota(jnp.int32, (e, fe), 1) % e).astype(jnp.float32)
  sum_embT = jax.lax.dot(sel, oT)
  order2 = 0.5 * (jnp.sum(sum_embT * sum_embT, axis=0, keepdims=True)
                  - jnp.sum(oT * oT, axis=0, keepdims=True))
  order1 = (jnp.sum(o1T_ref[...], axis=0, keepdims=True)
            + jax.lax.dot(w1d_ref[...], dT) + b1d_ref[...])
  ddT = jnp.maximum(jax.lax.dot(wdl_ref[...], dT) + bdl_ref[...], 0.0)
  dnnT = oT + ddT + order1 + order2
  h1 = jax.lax.dot(wl1_ref[...], dnnT) + bl1_ref[...]
  h1_ref[...] = h1
  st = jnp.concatenate([jnp.sum(h1, axis=1, keepdims=True),
                        jnp.sum(h1 * h1, axis=1, keepdims=True)], axis=1)

  @pl.when(i == 0)
  def _():
    st_ref[...] = st

  @pl.when(i > 0)
  def _():
    st_ref[...] = st_ref[...] + st


def _stage_mid_body(x_ref, st_in_ref, g_ref, be_ref, w_ref, bias_ref,
                    y_ref, st_ref, *, batch):
  i = pl.program_id(0)
  mean = st_in_ref[:, 0:1] / batch
  var = st_in_ref[:, 1:2] / batch - mean * mean
  a = g_ref[...] * lax.rsqrt(var + _EPS)
  c = be_ref[...] - mean * a
  x = jnp.maximum(x_ref[...] * a + c, 0.0)
  y = jax.lax.dot(w_ref[...], x) + bias_ref[...]
  y_ref[...] = y
  st = jnp.concatenate([jnp.sum(y, axis=1, keepdims=True),
                        jnp.sum(y * y, axis=1, keepdims=True)], axis=1)

  @pl.when(i == 0)
  def _():
    st_ref[...] = st

  @pl.when(i > 0)
  def _():
    st_ref[...] = st_ref[...] + st


def _stage_c_body(x_ref, st_in_ref, g_ref, be_ref, w_ref, bias_ref, out_ref,
                  *, batch):
  mean = st_in_ref[:, 0:1] / batch
  var = st_in_ref[:, 1:2] / batch - mean * mean
  a = g_ref[...] * lax.rsqrt(var + _EPS)
  c = be_ref[...] - mean * a
  x = jnp.maximum(x_ref[...] * a + c, 0.0)
  out_ref[...] = jax.lax.dot(w_ref[...], x) + bias_ref[...]


def _full(shape):
  return pl.BlockSpec(shape, lambda i: (0, 0))


def kernel(sparse, dense, order1_emb, order2_emb, W1d, b1d, Wdl, bdl, W_l1,
           b_l1, g1, be1, W_l2, b_l2, g2, be2, W_out, b_out):
  b, f = sparse.shape
  v, e = order2_emb.shape[1], order2_emb.shape[2]
  d = dense.shape[1]
  h1n, h2n = W_l1.shape[0], W_l2.shape[0]
  fe = f * e
  bb = 1024
  grid = (b // bb,)
  fl = jnp.float32

  o2T3, o1T = _sc_gather(jnp.transpose(order2_emb, (0, 2, 1)),
                         jnp.transpose(order1_emb, (0, 2, 1)), sparse.T)
  oT = o2T3.reshape(fe, b)

  h1T, st1 = pl.pallas_call(
      _stage_a_body,
      grid=grid,
      in_specs=[
          pl.BlockSpec((fe, bb), lambda i: (0, i)),
          pl.BlockSpec((f, bb), lambda i: (0, i)),
          pl.BlockSpec((d, bb), lambda i: (0, i)),
          _full((fe, d)), _full((fe, 1)), _full((1, d)), _full((1, 1)),
          _full((h1n, fe)), _full((h1n, 1)),
      ],
      out_specs=[pl.BlockSpec((h1n, bb), lambda i: (0, i)),
                 _full((h1n, 2))],
      out_shape=(jax.ShapeDtypeStruct((h1n, b), fl),
                 jax.ShapeDtypeStruct((h1n, 2), fl)),
  )(oT, o1T, dense.T, Wdl, bdl.reshape(fe, 1), W1d, b1d.reshape(1, 1),
    W_l1, b_l1.reshape(h1n, 1))

  h2T, st2 = pl.pallas_call(
      functools.partial(_stage_mid_body, batch=float(b)),
      grid=grid,
      in_specs=[
          pl.BlockSpec((h1n, bb), lambda i: (0, i)),
          _full((h1n, 2)), _full((h1n, 1)), _full((h1n, 1)),
          _full((h2n, h1n)), _full((h2n, 1)),
      ],
      out_specs=[pl.BlockSpec((h2n, bb), lambda i: (0, i)),
                 _full((h2n, 2))],
      out_shape=(jax.ShapeDtypeStruct((h2n, b), fl),
                 jax.ShapeDtypeStruct((h2n, 2), fl)),
  )(h1T, st1, g1.reshape(h1n, 1), be1.reshape(h1n, 1), W_l2,
    b_l2.reshape(h2n, 1))

  outT = pl.pallas_call(
      functools.partial(_stage_c_body, batch=float(b)),
      grid=grid,
      in_specs=[
          pl.BlockSpec((h2n, bb), lambda i: (0, i)),
          _full((h2n, 2)), _full((h2n, 1)), _full((h2n, 1)),
          _full((1, h2n)), _full((1, 1)),
      ],
      out_specs=pl.BlockSpec((1, bb), lambda i: (0, i)),
      out_shape=jax.ShapeDtypeStruct((1, b), fl),
  )(h2T, st2, g2.reshape(h2n, 1), be2.reshape(h2n, 1), W_out,
    b_out.reshape(1, 1))
  return outT.reshape(b, 1)
